# baseline (speedup 1.0000x reference)
import functools

import jax
import jax.numpy as jnp
import numpy as np
from jax.experimental import pallas as pl
from jax.experimental.pallas import tpu as pltpu

_N = 20000
_C = 81
_IMW = 1024.0
_CLIP = float(np.log(1000.0 / 16.0))
_R2D = float(180.0 / np.pi)


def _body(logits_ref, codes_ref, props_ref, boxes_ref, scores_ref):
    logits = logits_ref[...]
    m = jnp.max(logits, axis=0, keepdims=True)
    p = jnp.exp(logits - m)
    s = jnp.sum(p, axis=0, keepdims=True)
    scores_ref[...] = p / s

    codes = codes_ref[...]
    props = props_ref[...]
    cx = props[0:1, :]
    cy = props[1:2, :]
    w = props[2:3, :]
    h = props[3:4, :]
    a = props[4:5, :]

    nrow, c = codes.shape
    t = jax.lax.broadcasted_iota(jnp.int32, (nrow, c), 0) % 5
    is_xy = t < 2
    is_wh = (t == 2) | (t == 3)
    use_w = (t == 0) | (t == 2)
    use_h = (t == 1) | (t == 3)

    scale = jnp.where(is_xy, 0.1, jnp.where(is_wh, 0.2, 1.0))
    d = codes * scale
    e = jnp.exp(jnp.minimum(d, _CLIP))
    base = jnp.where(is_wh, e, d)
    mult = jnp.where(use_w, w, jnp.where(use_h, h, _R2D))
    addv = jnp.where(t == 0, cx, jnp.where(t == 1, cy, jnp.where(t == 4, a, 0.0)))
    out = base * mult + addv
    out = jnp.where(is_xy, jnp.clip(out, 0.0, _IMW - 1.0), out)
    boxes_ref[...] = out


@functools.partial(jax.jit, static_argnums=(3,))
def _run(class_logits, box_regression, proposals, block_cols):
    n = class_logits.shape[0]
    lg_t = jnp.transpose(class_logits)
    codes_t = jnp.transpose(box_regression)
    props_t = jnp.transpose(proposals)
    grid = (pl.cdiv(n, block_cols),)
    boxes_t, scores_t = pl.pallas_call(
        _body,
        grid=grid,
        in_specs=[
            pl.BlockSpec((_C, block_cols), lambda i: (0, i)),
            pl.BlockSpec((_C * 5, block_cols), lambda i: (0, i)),
            pl.BlockSpec((5, block_cols), lambda i: (0, i)),
        ],
        out_specs=[
            pl.BlockSpec((_C * 5, block_cols), lambda i: (0, i)),
            pl.BlockSpec((_C, block_cols), lambda i: (0, i)),
        ],
        out_shape=[
            jax.ShapeDtypeStruct((_C * 5, n), jnp.float32),
            jax.ShapeDtypeStruct((_C, n), jnp.float32),
        ],
        compiler_params=pltpu.CompilerParams(
            dimension_semantics=("parallel",),
        ),
    )(lg_t, codes_t, props_t)
    boxes = jnp.transpose(boxes_t).reshape(-1, 5)
    scores = jnp.transpose(scores_t).reshape(-1)
    return boxes, scores


def kernel(class_logits, box_regression, proposals, num_of_fwd_left=0):
    return _run(class_logits, box_regression, proposals, 2048)


# speedup vs baseline: 2.3923x; 2.3923x over previous
import functools

import jax
import jax.numpy as jnp
import numpy as np
from jax.experimental import pallas as pl
from jax.experimental.pallas import tpu as pltpu

_N = 20000
_C = 81
_IMW = 1024.0
_CLIP = float(np.log(1000.0 / 16.0))
_R2D = float(180.0 / np.pi)


def _body(logits_ref, codes_ref, props_ref, px_ref, py_ref, pw_ref, ph_ref,
          pa_ref, scores_ref):
    logits = logits_ref[...]
    m = jnp.max(logits, axis=0, keepdims=True)
    p = jnp.exp(logits - m)
    s = jnp.sum(p, axis=0, keepdims=True)
    scores_ref[...] = p / s

    codes = codes_ref[...]
    props = props_ref[...]
    cx = props[0:1, :]
    cy = props[1:2, :]
    w = props[2:3, :]
    h = props[3:4, :]
    a = props[4:5, :]

    row = jax.lax.broadcasted_iota(jnp.int32, (_C * 5, _C * 5), 0)
    col = jax.lax.broadcasted_iota(jnp.int32, (_C * 5, _C * 5), 1)
    perm = (col == 5 * (row % _C) + row // _C).astype(jnp.float32)
    deint = jax.lax.dot(perm, codes, preferred_element_type=jnp.float32)

    dx = deint[0:_C, :] * 0.1
    dy = deint[_C:2 * _C, :] * 0.1
    dw = jnp.minimum(deint[2 * _C:3 * _C, :] * 0.2, _CLIP)
    dh = jnp.minimum(deint[3 * _C:4 * _C, :] * 0.2, _CLIP)
    da = deint[4 * _C:5 * _C, :]

    px_ref[...] = jnp.clip(dx * w + cx, 0.0, _IMW - 1.0)
    py_ref[...] = jnp.clip(dy * h + cy, 0.0, _IMW - 1.0)
    pw_ref[...] = jnp.exp(dw) * w
    ph_ref[...] = jnp.exp(dh) * h
    pa_ref[...] = da * _R2D + a


@functools.partial(jax.jit, static_argnums=(3,))
def _run(class_logits, box_regression, proposals, block_cols):
    n = class_logits.shape[0]
    lg_t = jnp.transpose(class_logits)
    codes_t = jnp.transpose(box_regression)
    props_t = jnp.transpose(proposals)
    grid = (pl.cdiv(n, block_cols),)
    plane = pl.BlockSpec((_C, block_cols), lambda i: (0, i))
    plane_shape = jax.ShapeDtypeStruct((_C, n), jnp.float32)
    px, py, pw, ph, pa, scores_t = pl.pallas_call(
        _body,
        grid=grid,
        in_specs=[
            plane,
            pl.BlockSpec((_C * 5, block_cols), lambda i: (0, i)),
            pl.BlockSpec((5, block_cols), lambda i: (0, i)),
        ],
        out_specs=[plane] * 6,
        out_shape=[plane_shape] * 6,
        compiler_params=pltpu.CompilerParams(
            dimension_semantics=("parallel",),
        ),
    )(lg_t, codes_t, props_t)
    pred = jnp.stack(
        [jnp.transpose(px), jnp.transpose(py), jnp.transpose(pw),
         jnp.transpose(ph), jnp.transpose(pa)], axis=2)
    boxes = pred.reshape(-1, 5)
    scores = jnp.transpose(scores_t).reshape(-1)
    return boxes, scores


def kernel(class_logits, box_regression, proposals, num_of_fwd_left=0):
    return _run(class_logits, box_regression, proposals, 2048)


# speedup vs baseline: 2.8598x; 1.1954x over previous
import functools

import jax
import jax.numpy as jnp
import numpy as np
from jax.experimental import pallas as pl
from jax.experimental.pallas import tpu as pltpu

_N = 20000
_C = 81
_IMW = 1024.0
_CLIP = float(np.log(1000.0 / 16.0))
_R2D = float(180.0 / np.pi)


def _sel(j):
    row = jax.lax.broadcasted_iota(jnp.int32, (_C * 5, _C), 0)
    col = jax.lax.broadcasted_iota(jnp.int32, (_C * 5, _C), 1)
    return (row == 5 * col + j).astype(jnp.float32)


_DN_T = (((0,), (0,)), ((), ()))


def _body(logits_ref, codes_ref, props_ref, px_ref, py_ref, pw_ref, ph_ref,
          pa_ref, scores_ref):
    logits = logits_ref[...]
    m = jnp.max(logits, axis=0, keepdims=True)
    p = jnp.exp(logits - m)
    s = jnp.sum(p, axis=0, keepdims=True)
    prob = p / s
    eye = (jax.lax.broadcasted_iota(jnp.int32, (_C, _C), 0) ==
           jax.lax.broadcasted_iota(jnp.int32, (_C, _C), 1)).astype(jnp.float32)
    scores_ref[...] = jax.lax.dot_general(
        prob, eye, _DN_T, preferred_element_type=jnp.float32)

    codes = codes_ref[...]
    props = props_ref[...]

    def plane(j):
        return jax.lax.dot_general(
            codes, _sel(j), _DN_T, preferred_element_type=jnp.float32)

    eye5 = (jax.lax.broadcasted_iota(jnp.int32, (5, 5), 0) ==
            jax.lax.broadcasted_iota(jnp.int32, (5, 5), 1)).astype(jnp.float32)
    props_t = jax.lax.dot_general(
        props, eye5, _DN_T, preferred_element_type=jnp.float32)
    cx = props_t[:, 0:1]
    cy = props_t[:, 1:2]
    w = props_t[:, 2:3]
    h = props_t[:, 3:4]
    a = props_t[:, 4:5]

    px_ref[...] = jnp.clip(plane(0) * 0.1 * w + cx, 0.0, _IMW - 1.0)
    py_ref[...] = jnp.clip(plane(1) * 0.1 * h + cy, 0.0, _IMW - 1.0)
    pw_ref[...] = jnp.exp(jnp.minimum(plane(2) * 0.2, _CLIP)) * w
    ph_ref[...] = jnp.exp(jnp.minimum(plane(3) * 0.2, _CLIP)) * h
    pa_ref[...] = plane(4) * _R2D + a


@functools.partial(jax.jit, static_argnums=(3,))
def _run(class_logits, box_regression, proposals, block_cols):
    n = class_logits.shape[0]
    lg_t = jnp.transpose(class_logits)
    codes_t = jnp.transpose(box_regression)
    props_t = jnp.transpose(proposals)
    grid = (pl.cdiv(n, block_cols),)
    out_spec = pl.BlockSpec((block_cols, _C), lambda i: (i, 0))
    out_shape = jax.ShapeDtypeStruct((n, _C), jnp.float32)
    px, py, pw, ph, pa, scores_rm = pl.pallas_call(
        _body,
        grid=grid,
        in_specs=[
            pl.BlockSpec((_C, block_cols), lambda i: (0, i)),
            pl.BlockSpec((_C * 5, block_cols), lambda i: (0, i)),
            pl.BlockSpec((5, block_cols), lambda i: (0, i)),
        ],
        out_specs=[out_spec] * 6,
        out_shape=[out_shape] * 6,
        compiler_params=pltpu.CompilerParams(
            dimension_semantics=("parallel",),
        ),
    )(lg_t, codes_t, props_t)
    pred = jnp.stack([px, py, pw, ph, pa], axis=2)
    boxes = pred.reshape(-1, 5)
    scores = scores_rm.reshape(-1)
    return boxes, scores


def kernel(class_logits, box_regression, proposals, num_of_fwd_left=0):
    return _run(class_logits, box_regression, proposals, 2048)
